# baseline (speedup 1.0000x reference)
import functools

import jax
import jax.numpy as jnp
from jax import lax
from jax.experimental import pallas as pl
from jax.experimental.pallas import tpu as pltpu
from jax.experimental.pallas import tpu_sc as plsc

NC = 2
NS = 16
NW = NC * NS
B = 204800
D = 64
BPW = B // NW
CHUNK = 128
NCHUNK = BPW // CHUNK
NCOL = B // CHUNK
CPAD = CHUNK + 1

_mesh = plsc.VectorSubcoreMesh(core_axis_name="c", subcore_axis_name="s")


def _transpose_block(g_v, gt_v, scatter_idx):
    @plsc.parallel_loop(0, CHUNK, unroll=4)
    def _(i):
        ci = jnp.broadcast_to(i, (16,)).astype(jnp.int32)
        for k, (rv, zv, cv) in enumerate(scatter_idx):
            v = g_v[i, pl.ds(16 * k, 16)]
            plsc.store_scatter(gt_v, [rv, zv, cv, ci], v)


@functools.partial(
    pl.kernel,
    mesh=_mesh,
    out_type=jax.ShapeDtypeStruct((8, NCOL, 8, CHUNK), jnp.float32),
    scratch_types=[
        pltpu.VMEM((NCHUNK, CHUNK), jnp.int32),
        pltpu.VMEM((CHUNK, D), jnp.float32),
        pltpu.VMEM((CHUNK, D), jnp.float32),
        pltpu.VMEM((8, 1, 8, CPAD), jnp.float32),
        pltpu.VMEM((8, 1, 8, CPAD), jnp.float32),
        pltpu.SemaphoreType.DMA((2,)),
        pltpu.SemaphoreType.DMA((2,)),
    ],
    compiler_params=pltpu.CompilerParams(
        use_tc_tiling_on_sc=False, needs_layout_passes=False
    ),
)
def _gather_kernel(idx_hbm, table_hbm, y_hbm, idx_v, g0, g1, gt0, gt1, gsem, wsem):
    wid = lax.axis_index("s") * NC + lax.axis_index("c")
    cb = wid * NCHUNK
    pltpu.sync_copy(idx_hbm.at[wid], idx_v)

    lanes = jax.lax.iota(jnp.int32, 16)
    zv = jnp.zeros((16,), jnp.int32)
    scatter_idx = []
    for k in range(D // 16):
        dv = lanes + (16 * k)
        scatter_idx.append(
            (lax.shift_right_logical(dv, 3), zv, lax.bitwise_and(dv, 7))
        )

    def gt_src(gt_v):
        return gt_v.at[:, :, :, pl.ds(0, CHUNK)]

    pltpu.async_copy(table_hbm.at[idx_v.at[0]], g0, gsem.at[0])
    pltpu.async_copy(table_hbm.at[idx_v.at[1]], g1, gsem.at[1])

    def step(t, carry):
        for half, g_v, gt_v in ((0, g0, gt0), (1, g1, gt1)):
            k = 2 * t + half
            pltpu.make_async_copy(table_hbm.at[idx_v.at[k]], g_v, gsem.at[half]).wait()

            @pl.when(t > 0)
            def _():
                pltpu.make_async_copy(
                    gt_src(gt_v), y_hbm.at[:, pl.ds(cb + k - 2, 1)], wsem.at[half]
                ).wait()

            _transpose_block(g_v, gt_v, scatter_idx)
            pltpu.async_copy(gt_src(gt_v), y_hbm.at[:, pl.ds(cb + k, 1)], wsem.at[half])

            @pl.when(t < NCHUNK // 2 - 1)
            def _():
                pltpu.async_copy(
                    table_hbm.at[idx_v.at[k + 2]], g_v, gsem.at[half]
                )

        return carry

    lax.fori_loop(0, NCHUNK // 2, step, 0)

    pltpu.make_async_copy(gt_src(gt0), y_hbm.at[:, pl.ds(cb + NCHUNK - 2, 1)], wsem.at[0]).wait()
    pltpu.make_async_copy(gt_src(gt1), y_hbm.at[:, pl.ds(cb + NCHUNK - 1, 1)], wsem.at[1]).wait()


def kernel(indices, table):
    idx = indices.astype(jnp.int32).reshape(NW, NCHUNK, CHUNK)
    y = _gather_kernel(idx, table)
    return y.transpose(1, 3, 0, 2).reshape(B, D)


# speedup vs baseline: 1.0861x; 1.0861x over previous
import functools

import jax
import jax.numpy as jnp
from jax import lax
from jax.experimental import pallas as pl
from jax.experimental.pallas import tpu as pltpu
from jax.experimental.pallas import tpu_sc as plsc

NC = 2
NS = 16
NW = NC * NS
B = 204800
D = 64
BPW = B // NW
CHUNK = 128
NCHUNK = BPW // CHUNK
NCOL = B // CHUNK
CPAD = CHUNK + 1
NBUF = 5

_mesh = plsc.VectorSubcoreMesh(core_axis_name="c", subcore_axis_name="s")


def _transpose_block(g_v, gt_v, scatter_idx):
    @plsc.parallel_loop(0, CHUNK, unroll=4)
    def _(i):
        ci = jnp.broadcast_to(i, (16,)).astype(jnp.int32)
        for k, (rv, zv, cv) in enumerate(scatter_idx):
            v = g_v[i, pl.ds(16 * k, 16)]
            plsc.store_scatter(gt_v, [rv, zv, cv, ci], v)


@functools.partial(
    pl.kernel,
    mesh=_mesh,
    out_type=jax.ShapeDtypeStruct((8, NCOL, 8, CHUNK), jnp.float32),
    scratch_types=[
        pltpu.VMEM((NCHUNK, CHUNK), jnp.int32),
        pltpu.VMEM((NBUF, CHUNK, D), jnp.float32),
        pltpu.VMEM((NBUF, 8, 1, 8, CPAD), jnp.float32),
        pltpu.SemaphoreType.DMA((NBUF,)),
        pltpu.SemaphoreType.DMA((NBUF,)),
    ],
    compiler_params=pltpu.CompilerParams(
        use_tc_tiling_on_sc=False, needs_layout_passes=False
    ),
)
def _gather_kernel(idx_hbm, table_hbm, y_hbm, idx_v, g_v, gt_v, gsem, wsem):
    wid = lax.axis_index("s") * NC + lax.axis_index("c")
    cb = wid * NCHUNK
    pltpu.sync_copy(idx_hbm.at[wid], idx_v)

    lanes = jax.lax.iota(jnp.int32, 16)
    zv = jnp.zeros((16,), jnp.int32)
    scatter_idx = []
    for k in range(D // 16):
        dv = lanes + (16 * k)
        scatter_idx.append(
            (lax.shift_right_logical(dv, 3), zv, lax.bitwise_and(dv, 7))
        )

    def gt_src(b):
        return gt_v.at[b, :, :, :, pl.ds(0, CHUNK)]

    for b in range(NBUF):
        pltpu.async_copy(table_hbm.at[idx_v.at[b]], g_v.at[b], gsem.at[b])

    def step(t, carry):
        for b in range(NBUF):
            k = NBUF * t + b
            pltpu.make_async_copy(
                table_hbm.at[idx_v.at[k]], g_v.at[b], gsem.at[b]
            ).wait()

            @pl.when(t > 0)
            def _():
                pltpu.make_async_copy(
                    gt_src(b), y_hbm.at[:, pl.ds(cb + k - NBUF, 1)], wsem.at[b]
                ).wait()

            _transpose_block(g_v.at[b], gt_v.at[b], scatter_idx)
            pltpu.async_copy(gt_src(b), y_hbm.at[:, pl.ds(cb + k, 1)], wsem.at[b])

            @pl.when(t < NCHUNK // NBUF - 1)
            def _():
                pltpu.async_copy(
                    table_hbm.at[idx_v.at[k + NBUF]], g_v.at[b], gsem.at[b]
                )

        return carry

    lax.fori_loop(0, NCHUNK // NBUF, step, 0)

    for b in range(NBUF):
        pltpu.make_async_copy(
            gt_src(b), y_hbm.at[:, pl.ds(cb + NCHUNK - NBUF + b, 1)], wsem.at[b]
        ).wait()


def kernel(indices, table):
    idx = indices.astype(jnp.int32).reshape(NW, NCHUNK, CHUNK)
    y = _gather_kernel(idx, table)
    return y.transpose(1, 3, 0, 2).reshape(B, D)
